# baseline (speedup 1.0000x reference)
import functools

import jax
import jax.numpy as jnp
from jax.experimental import pallas as pl
from jax.experimental.pallas import tpu as pltpu
from jax.experimental.pallas import tpu_sc as plsc

_B = 16
_T = 256
_D = 256
_NC = 256
_BETA = 0.25
_BM = 4096
_GRID = (_B * _T) // _BM
_TH = _T // 2


def _dist_body(z_ref, w_ref, d_ref):
    zb = z_ref[...]
    w = w_ref[...]
    s_z = jnp.sum(zb * zb, axis=1, keepdims=True)
    s_w = jnp.sum(w * w, axis=1)
    c = jax.lax.dot_general(zb, w, (((1,), (1,)), ((), ())),
                            preferred_element_type=jnp.float32)
    d_ref[...] = (s_z + s_w[None, :]) - 2.0 * c


_dist = pl.pallas_call(
    _dist_body,
    grid=(_GRID,),
    in_specs=[
        pl.BlockSpec((_BM, _D), lambda i: (i, 0)),
        pl.BlockSpec((_NC, _D), lambda i: (0, 0)),
    ],
    out_specs=pl.BlockSpec((_BM, _NC), lambda i: (i, 0)),
    out_shape=jax.ShapeDtypeStruct((_B * _T, _NC), jnp.float32),
)


def _scan_body(d_hbm, ind_hbm, stats_hbm,
               d_v, ind_a, ind_b, s_v, sem_in):
    c = jax.lax.axis_index("c")
    s = jax.lax.axis_index("s")
    wid = s * 2 + c

    @pl.when(wid < _B)
    def _():
        cp_in = pltpu.async_copy(
            d_hbm.at[wid, pl.ds(_TH, _T - _TH)], d_v.at[pl.ds(_TH, _T - _TH)],
            sem_in)
        pltpu.sync_copy(d_hbm.at[wid, pl.ds(0, _TH)], d_v.at[pl.ds(0, _TH)])
        lanes = jax.lax.iota(jnp.int32, 16)
        zeros = jnp.zeros((16,), jnp.int32)

        def step(t, ind, accl, acc, k):
            row = jnp.full((16,), t, jnp.int32)
            d_here = plsc.load_gather(d_v, [row, ind])
            d_next = plsc.load_gather(d_v, [row, ind + 1])
            le = d_here <= d_next
            ind = jnp.where(le, ind, ind + 1)
            accl = accl + jnp.where(le, d_here, d_next)
            acc = jnp.where(lanes == k, ind, acc)
            return ind, accl, acc

        accl = plsc.load_gather(d_v, [zeros, zeros])
        ind = zeros
        acc = zeros
        for k in range(1, 16):
            ind, accl, acc = step(jnp.int32(k), ind, accl, acc, k)
        ind_a[pl.ds(0, 16)] = acc

        def outer_a(g, carry):
            ind, accl = carry
            base = g * 16
            acc = jnp.zeros((16,), jnp.int32)
            for k in range(16):
                ind, accl, acc = step(base + k, ind, accl, acc, k)
            ind_a[pl.ds(base, 16)] = acc
            return ind, accl

        def outer_b(g, carry):
            ind, accl = carry
            base = g * 16
            acc = jnp.zeros((16,), jnp.int32)
            for k in range(16):
                ind, accl, acc = step(base + k, ind, accl, acc, k)
            ind_b[pl.ds(base - _TH, 16)] = acc
            return ind, accl

        ind, accl = jax.lax.fori_loop(1, _TH // 16, outer_a, (ind, accl))
        pltpu.sync_copy(ind_a, ind_hbm.at[wid, pl.ds(0, _TH)])
        cp_in.wait()
        ind, accl = jax.lax.fori_loop(_TH // 16, _T // 16, outer_b,
                                      (ind, accl))
        pltpu.sync_copy(ind_b, ind_hbm.at[wid, pl.ds(_TH, _T - _TH)])
        s_v[...] = accl
        pltpu.sync_copy(s_v, stats_hbm.at[wid])


_scan = functools.partial(
    pl.kernel,
    out_type=[
        jax.ShapeDtypeStruct((_B, _T), jnp.int32),
        jax.ShapeDtypeStruct((_B, 16), jnp.float32),
    ],
    mesh=plsc.VectorSubcoreMesh(core_axis_name="c", subcore_axis_name="s"),
    compiler_params=pltpu.CompilerParams(
        use_tc_tiling_on_sc=False, needs_layout_passes=False),
    scratch_types=[
        pltpu.VMEM((_T, _NC), jnp.float32),
        pltpu.VMEM((_TH,), jnp.int32),
        pltpu.VMEM((_TH,), jnp.int32),
        pltpu.VMEM((16,), jnp.float32),
        pltpu.SemaphoreType.DMA,
    ],
)(_scan_body)


def _zq_body(ind_ref, w_ref, st_ref, zq_ref, loss_ref, v_ref):
    pid = pl.program_id(0)
    ind = ind_ref[...]
    w = w_ref[...]
    iot = jax.lax.broadcasted_iota(jnp.int32, (_BM, _NC), 1)
    oh = jnp.where(iot == ind, 1.0, 0.0).astype(jnp.float32)
    zq_ref[...] = jax.lax.dot_general(oh, w, (((1,), (0,)), ((), ())),
                                      precision=jax.lax.Precision.HIGHEST,
                                      preferred_element_type=jnp.float32)
    pmax = jnp.max(ind)

    @pl.when(pid == 0)
    def _():
        v_ref[0, 0] = pmax

    @pl.when(pid != 0)
    def _():
        v_ref[0, 0] = jnp.maximum(v_ref[0, 0], pmax)

    @pl.when(pid == _GRID - 1)
    def _():
        m = jnp.sum(st_ref[:, 0:1]) * jnp.float32(1.0 / (_B * _T * _D))
        loss_ref[0, 0] = jnp.float32(_BETA) * m + m


_zq = pl.pallas_call(
    _zq_body,
    grid=(_GRID,),
    in_specs=[
        pl.BlockSpec((_BM, 1), lambda i: (i, 0)),
        pl.BlockSpec((_NC, _D), lambda i: (0, 0)),
        pl.BlockSpec((_B, 16), lambda i: (0, 0)),
    ],
    out_specs=[
        pl.BlockSpec((_BM, _D), lambda i: (i, 0)),
        pl.BlockSpec((1, 1), lambda i: (0, 0), memory_space=pltpu.SMEM),
        pl.BlockSpec((1, 1), lambda i: (0, 0), memory_space=pltpu.SMEM),
    ],
    out_shape=[
        jax.ShapeDtypeStruct((_B * _T, _D), jnp.float32),
        jax.ShapeDtypeStruct((1, 1), jnp.float32),
        jax.ShapeDtypeStruct((1, 1), jnp.int32),
    ],
)


def kernel(z, weight):
    zf = z.reshape(_B * _T, _D)
    d = _dist(zf, weight)
    ind, stats = _scan(d.reshape(_B, _T, _NC))
    zq, loss, vmax = _zq(ind.reshape(_B * _T, 1), weight, stats)
    return (zq.reshape(z.shape), loss.reshape(()), ind, vmax.reshape(()))


# speedup vs baseline: 1.0335x; 1.0335x over previous
import functools

import jax
import jax.numpy as jnp
from jax.experimental import pallas as pl
from jax.experimental.pallas import tpu as pltpu
from jax.experimental.pallas import tpu_sc as plsc

_B = 16
_T = 256
_D = 256
_NC = 256
_BETA = 0.25
_BM = 2048
_GRID = (_B * _T) // _BM
_TH = _T // 2


def _dist_body(z_ref, w_ref, d_ref):
    zb = z_ref[...]
    w = w_ref[...]
    s_z = jnp.sum(zb * zb, axis=1, keepdims=True)
    s_w = jnp.sum(w * w, axis=1)
    c = jax.lax.dot_general(zb, w, (((1,), (1,)), ((), ())),
                            preferred_element_type=jnp.float32)
    d_ref[...] = (s_z + s_w[None, :]) - 2.0 * c


_dist = pl.pallas_call(
    _dist_body,
    grid=(_GRID,),
    in_specs=[
        pl.BlockSpec((_BM, _D), lambda i: (i, 0)),
        pl.BlockSpec((_NC, _D), lambda i: (0, 0)),
    ],
    out_specs=pl.BlockSpec((_BM, _NC), lambda i: (i, 0)),
    out_shape=jax.ShapeDtypeStruct((_B * _T, _NC), jnp.float32),
)


def _scan_body(d_hbm, ind_hbm, stats_hbm,
               d_v, ind_a, ind_b, s_v, sem0, sem1, sem2):
    c = jax.lax.axis_index("c")
    s = jax.lax.axis_index("s")
    wid = s * 2 + c

    @pl.when(wid < _B)
    def _():
        cp0 = pltpu.async_copy(
            d_hbm.at[wid, pl.ds(0, 16)], d_v.at[pl.ds(0, 16)], sem0)
        cp1 = pltpu.async_copy(
            d_hbm.at[wid, pl.ds(16, _TH - 16)], d_v.at[pl.ds(16, _TH - 16)],
            sem1)
        cp_in = pltpu.async_copy(
            d_hbm.at[wid, pl.ds(_TH, _T - _TH)], d_v.at[pl.ds(_TH, _T - _TH)],
            sem2)
        cp0.wait()
        lanes = jax.lax.iota(jnp.int32, 16)
        zeros = jnp.zeros((16,), jnp.int32)

        def step(t, ind, accl, acc, k):
            row = jnp.full((16,), t, jnp.int32)
            d_here = plsc.load_gather(d_v, [row, ind])
            d_next = plsc.load_gather(d_v, [row, ind + 1])
            le = d_here <= d_next
            ind = jnp.where(le, ind, ind + 1)
            accl = accl + jnp.where(le, d_here, d_next)
            acc = jnp.where(lanes == k, ind, acc)
            return ind, accl, acc

        accl = plsc.load_gather(d_v, [zeros, zeros])
        ind = zeros
        acc = zeros
        for k in range(1, 16):
            ind, accl, acc = step(jnp.int32(k), ind, accl, acc, k)
        ind_a[pl.ds(0, 16)] = acc

        def outer_a(g, carry):
            ind, accl = carry
            base = g * 16
            acc = jnp.zeros((16,), jnp.int32)
            for k in range(16):
                ind, accl, acc = step(base + k, ind, accl, acc, k)
            ind_a[pl.ds(base, 16)] = acc
            return ind, accl

        def outer_b(g, carry):
            ind, accl = carry
            base = g * 16
            acc = jnp.zeros((16,), jnp.int32)
            for k in range(16):
                ind, accl, acc = step(base + k, ind, accl, acc, k)
            ind_b[pl.ds(base - _TH, 16)] = acc
            return ind, accl

        cp1.wait()
        ind, accl = jax.lax.fori_loop(1, _TH // 16, outer_a, (ind, accl))
        pltpu.sync_copy(ind_a, ind_hbm.at[wid, pl.ds(0, _TH)])
        cp_in.wait()
        ind, accl = jax.lax.fori_loop(_TH // 16, _T // 16, outer_b,
                                      (ind, accl))
        pltpu.sync_copy(ind_b, ind_hbm.at[wid, pl.ds(_TH, _T - _TH)])
        s_v[...] = accl
        pltpu.sync_copy(s_v, stats_hbm.at[wid])


_scan = functools.partial(
    pl.kernel,
    out_type=[
        jax.ShapeDtypeStruct((_B, _T), jnp.int32),
        jax.ShapeDtypeStruct((_B, 16), jnp.float32),
    ],
    mesh=plsc.VectorSubcoreMesh(core_axis_name="c", subcore_axis_name="s"),
    compiler_params=pltpu.CompilerParams(
        use_tc_tiling_on_sc=False, needs_layout_passes=False),
    scratch_types=[
        pltpu.VMEM((_T, _NC), jnp.float32),
        pltpu.VMEM((_TH,), jnp.int32),
        pltpu.VMEM((_TH,), jnp.int32),
        pltpu.VMEM((16,), jnp.float32),
        pltpu.SemaphoreType.DMA,
        pltpu.SemaphoreType.DMA,
        pltpu.SemaphoreType.DMA,
    ],
)(_scan_body)


def _zq_body(ind_ref, w_ref, st_ref, zq_ref, loss_ref, v_ref):
    pid = pl.program_id(0)
    ind = ind_ref[...]
    w = w_ref[...]
    iot = jax.lax.broadcasted_iota(jnp.int32, (_BM, _NC), 1)
    oh = jnp.where(iot == ind, 1.0, 0.0).astype(jnp.float32)
    zq_ref[...] = jax.lax.dot_general(oh, w, (((1,), (0,)), ((), ())),
                                      precision=jax.lax.Precision.HIGHEST,
                                      preferred_element_type=jnp.float32)
    pmax = jnp.max(ind)

    @pl.when(pid == 0)
    def _():
        v_ref[0, 0] = pmax

    @pl.when(pid != 0)
    def _():
        v_ref[0, 0] = jnp.maximum(v_ref[0, 0], pmax)

    @pl.when(pid == _GRID - 1)
    def _():
        m = jnp.sum(st_ref[:, 0:1]) * jnp.float32(1.0 / (_B * _T * _D))
        loss_ref[0, 0] = jnp.float32(_BETA) * m + m


_zq = pl.pallas_call(
    _zq_body,
    grid=(_GRID,),
    in_specs=[
        pl.BlockSpec((_BM, 1), lambda i: (i, 0)),
        pl.BlockSpec((_NC, _D), lambda i: (0, 0)),
        pl.BlockSpec((_B, 16), lambda i: (0, 0)),
    ],
    out_specs=[
        pl.BlockSpec((_BM, _D), lambda i: (i, 0)),
        pl.BlockSpec((1, 1), lambda i: (0, 0), memory_space=pltpu.SMEM),
        pl.BlockSpec((1, 1), lambda i: (0, 0), memory_space=pltpu.SMEM),
    ],
    out_shape=[
        jax.ShapeDtypeStruct((_B * _T, _D), jnp.float32),
        jax.ShapeDtypeStruct((1, 1), jnp.float32),
        jax.ShapeDtypeStruct((1, 1), jnp.int32),
    ],
)


def kernel(z, weight):
    zf = z.reshape(_B * _T, _D)
    d = _dist(zf, weight)
    ind, stats = _scan(d.reshape(_B, _T, _NC))
    zq, loss, vmax = _zq(ind.reshape(_B * _T, 1), weight, stats)
    return (zq.reshape(z.shape), loss.reshape(()), ind, vmax.reshape(()))


# speedup vs baseline: 1.0652x; 1.0307x over previous
import functools

import jax
import jax.numpy as jnp
from jax.experimental import pallas as pl
from jax.experimental.pallas import tpu as pltpu
from jax.experimental.pallas import tpu_sc as plsc

_B = 16
_T = 256
_D = 256
_NC = 256
_BETA = 0.25
_BM = 2048
_GRID = (_B * _T) // _BM
_TH = _T // 2


def _dist_body(z_ref, w_ref, d_ref):
    zb = z_ref[...]
    w = w_ref[...]
    s_z = jnp.sum(zb * zb, axis=1, keepdims=True)
    s_w = jnp.sum(w * w, axis=1)
    c = jax.lax.dot_general(zb, w, (((1,), (1,)), ((), ())),
                            preferred_element_type=jnp.float32)
    d_ref[...] = (s_z + s_w[None, :]) - 2.0 * c


_dist = pl.pallas_call(
    _dist_body,
    grid=(_GRID,),
    in_specs=[
        pl.BlockSpec((_BM, _D), lambda i: (i, 0)),
        pl.BlockSpec((_NC, _D), lambda i: (0, 0)),
    ],
    out_specs=pl.BlockSpec((_BM, _NC), lambda i: (i, 0)),
    out_shape=jax.ShapeDtypeStruct((_B * _T, _NC), jnp.float32),
)


def _scan_body(d_hbm, ind_hbm, stats_hbm,
               d_v, ind_a, ind_b, s_v, sem0, sem1, sem2):
    c = jax.lax.axis_index("c")
    s = jax.lax.axis_index("s")
    wid = s + 0 * c

    @pl.when(wid < _B)
    def _():
        cp0 = pltpu.async_copy(
            d_hbm.at[wid, pl.ds(0, 16)], d_v.at[pl.ds(0, 16)], sem0)
        cp1 = pltpu.async_copy(
            d_hbm.at[wid, pl.ds(16, _TH - 16)], d_v.at[pl.ds(16, _TH - 16)],
            sem1)
        cp_in = pltpu.async_copy(
            d_hbm.at[wid, pl.ds(_TH, _T - _TH)], d_v.at[pl.ds(_TH, _T - _TH)],
            sem2)
        cp0.wait()
        lanes = jax.lax.iota(jnp.int32, 16)
        zeros = jnp.zeros((16,), jnp.int32)

        def step(t, ind, accl, acc, k):
            row = jnp.full((16,), t, jnp.int32)
            d_here = plsc.load_gather(d_v, [row, ind])
            d_next = plsc.load_gather(d_v, [row, ind + 1])
            le = d_here <= d_next
            ind = jnp.where(le, ind, ind + 1)
            accl = accl + jnp.where(le, d_here, d_next)
            acc = jnp.where(lanes == k, ind, acc)
            return ind, accl, acc

        accl = plsc.load_gather(d_v, [zeros, zeros])
        ind = zeros
        acc = zeros
        for k in range(1, 16):
            ind, accl, acc = step(jnp.int32(k), ind, accl, acc, k)
        ind_a[pl.ds(0, 16)] = acc

        def outer_a(g, carry):
            ind, accl = carry
            base = g * 16
            acc = jnp.zeros((16,), jnp.int32)
            for k in range(16):
                ind, accl, acc = step(base + k, ind, accl, acc, k)
            ind_a[pl.ds(base, 16)] = acc
            return ind, accl

        def outer_b(g, carry):
            ind, accl = carry
            base = g * 16
            acc = jnp.zeros((16,), jnp.int32)
            for k in range(16):
                ind, accl, acc = step(base + k, ind, accl, acc, k)
            ind_b[pl.ds(base - _TH, 16)] = acc
            return ind, accl

        cp1.wait()
        ind, accl = jax.lax.fori_loop(1, _TH // 16, outer_a, (ind, accl))
        pltpu.sync_copy(ind_a, ind_hbm.at[wid, pl.ds(0, _TH)])
        cp_in.wait()
        ind, accl = jax.lax.fori_loop(_TH // 16, _T // 16, outer_b,
                                      (ind, accl))
        pltpu.sync_copy(ind_b, ind_hbm.at[wid, pl.ds(_TH, _T - _TH)])
        s_v[...] = accl
        pltpu.sync_copy(s_v, stats_hbm.at[wid])


_scan = functools.partial(
    pl.kernel,
    out_type=[
        jax.ShapeDtypeStruct((_B, _T), jnp.int32),
        jax.ShapeDtypeStruct((_B, 16), jnp.float32),
    ],
    mesh=plsc.VectorSubcoreMesh(core_axis_name="c", subcore_axis_name="s",
                                num_cores=1),
    compiler_params=pltpu.CompilerParams(
        use_tc_tiling_on_sc=False, needs_layout_passes=False),
    scratch_types=[
        pltpu.VMEM((_T, _NC), jnp.float32),
        pltpu.VMEM((_TH,), jnp.int32),
        pltpu.VMEM((_TH,), jnp.int32),
        pltpu.VMEM((16,), jnp.float32),
        pltpu.SemaphoreType.DMA,
        pltpu.SemaphoreType.DMA,
        pltpu.SemaphoreType.DMA,
    ],
)(_scan_body)


def _zq_body(ind_ref, w_ref, st_ref, zq_ref, loss_ref, v_ref):
    pid = pl.program_id(0)
    ind = ind_ref[...]
    w = w_ref[...]
    iot = jax.lax.broadcasted_iota(jnp.int32, (_BM, _NC), 1)
    oh = jnp.where(iot == ind, 1.0, 0.0).astype(jnp.float32)
    zq_ref[...] = jax.lax.dot_general(oh, w, (((1,), (0,)), ((), ())),
                                      precision=jax.lax.Precision.HIGHEST,
                                      preferred_element_type=jnp.float32)
    pmax = jnp.max(ind)

    @pl.when(pid == 0)
    def _():
        v_ref[0, 0] = pmax

    @pl.when(pid != 0)
    def _():
        v_ref[0, 0] = jnp.maximum(v_ref[0, 0], pmax)

    @pl.when(pid == _GRID - 1)
    def _():
        m = jnp.sum(st_ref[:, 0:1]) * jnp.float32(1.0 / (_B * _T * _D))
        loss_ref[0, 0] = jnp.float32(_BETA) * m + m


_zq = pl.pallas_call(
    _zq_body,
    grid=(_GRID,),
    in_specs=[
        pl.BlockSpec((_BM, 1), lambda i: (i, 0)),
        pl.BlockSpec((_NC, _D), lambda i: (0, 0)),
        pl.BlockSpec((_B, 16), lambda i: (0, 0)),
    ],
    out_specs=[
        pl.BlockSpec((_BM, _D), lambda i: (i, 0)),
        pl.BlockSpec((1, 1), lambda i: (0, 0), memory_space=pltpu.SMEM),
        pl.BlockSpec((1, 1), lambda i: (0, 0), memory_space=pltpu.SMEM),
    ],
    out_shape=[
        jax.ShapeDtypeStruct((_B * _T, _D), jnp.float32),
        jax.ShapeDtypeStruct((1, 1), jnp.float32),
        jax.ShapeDtypeStruct((1, 1), jnp.int32),
    ],
)


def kernel(z, weight):
    zf = z.reshape(_B * _T, _D)
    d = _dist(zf, weight)
    ind, stats = _scan(d.reshape(_B, _T, _NC))
    zq, loss, vmax = _zq(ind.reshape(_B * _T, 1), weight, stats)
    return (zq.reshape(z.shape), loss.reshape(()), ind, vmax.reshape(()))


# speedup vs baseline: 1.0811x; 1.0150x over previous
import functools

import jax
import jax.numpy as jnp
from jax.experimental import pallas as pl
from jax.experimental.pallas import tpu as pltpu
from jax.experimental.pallas import tpu_sc as plsc

_B = 16
_T = 256
_D = 256
_NC = 256
_BETA = 0.25
_BM = 2048
_GRID = (_B * _T) // _BM
_TH = _T // 2


def _dist_body(z_ref, w_ref, d_ref):
    zb = z_ref[...]
    w = w_ref[...]
    s_z = jnp.sum(zb * zb, axis=1, keepdims=True)
    s_w = jnp.sum(w * w, axis=1)
    c = jax.lax.dot_general(zb, w, (((1,), (1,)), ((), ())),
                            preferred_element_type=jnp.float32)
    d_ref[...] = (s_z + s_w[None, :]) - 2.0 * c


_dist = pl.pallas_call(
    _dist_body,
    grid=(_GRID,),
    in_specs=[
        pl.BlockSpec((_BM, _D), lambda i: (i, 0)),
        pl.BlockSpec((_NC, _D), lambda i: (0, 0)),
    ],
    out_specs=pl.BlockSpec((_BM, _NC), lambda i: (i, 0)),
    out_shape=jax.ShapeDtypeStruct((_B * _T, _NC), jnp.float32),
)


def _scan_body(d_hbm, ind_hbm, stats_hbm,
               d_v, ind_a, ind_b, s_v, sem0, sem1, sem2):
    c = jax.lax.axis_index("c")
    s = jax.lax.axis_index("s")
    wid = s + 0 * c

    @pl.when(wid < _B)
    def _():
        cp0 = pltpu.async_copy(
            d_hbm.at[wid, pl.ds(0, 48)], d_v.at[pl.ds(0, 48)], sem0)
        cp1 = pltpu.async_copy(
            d_hbm.at[wid, pl.ds(48, _TH - 48)], d_v.at[pl.ds(48, _TH - 48)],
            sem1)
        cp_in = pltpu.async_copy(
            d_hbm.at[wid, pl.ds(_TH, _T - _TH)], d_v.at[pl.ds(_TH, _T - _TH)],
            sem2)
        cp0.wait()
        lanes = jax.lax.iota(jnp.int32, 16)
        zeros = jnp.zeros((16,), jnp.int32)
        ones = jnp.full((16,), 1, jnp.int32)

        def step(t, ind, h0, h1, accl, acc, k):
            row_n = jnp.full((16,), t + 1, jnp.int32)
            v0 = plsc.load_gather(d_v, [row_n, ind])
            v1 = plsc.load_gather(d_v, [row_n, ind + 1])
            v2 = plsc.load_gather(d_v, [row_n, ind + 2])
            le = h0 <= h1
            ind = jnp.where(le, ind, ind + 1)
            accl = accl + jnp.where(le, h0, h1)
            acc = jnp.where(lanes == k, ind, acc)
            h0 = jnp.where(le, v0, v1)
            h1 = jnp.where(le, v1, v2)
            return ind, h0, h1, accl, acc

        accl = plsc.load_gather(d_v, [zeros, zeros])
        ind = zeros
        acc = zeros
        h0 = plsc.load_gather(d_v, [ones, zeros])
        h1 = plsc.load_gather(d_v, [ones, ones])
        for k in range(1, 16):
            ind, h0, h1, accl, acc = step(jnp.int32(k), ind, h0, h1,
                                          accl, acc, k)
        ind_a[pl.ds(0, 16)] = acc

        def outer_a(g, carry):
            ind, h0, h1, accl = carry
            base = g * 16
            acc = jnp.zeros((16,), jnp.int32)
            for k in range(16):
                ind, h0, h1, accl, acc = step(base + k, ind, h0, h1,
                                              accl, acc, k)
            ind_a[pl.ds(base, 16)] = acc
            return ind, h0, h1, accl

        def outer_b(g, carry):
            ind, h0, h1, accl = carry
            base = g * 16
            acc = jnp.zeros((16,), jnp.int32)
            for k in range(16):
                ind, h0, h1, accl, acc = step(base + k, ind, h0, h1,
                                              accl, acc, k)
            ind_b[pl.ds(base - _TH, 16)] = acc
            return ind, h0, h1, accl

        carry = (ind, h0, h1, accl)
        carry = jax.lax.fori_loop(1, 2, outer_a, carry)
        cp1.wait()
        carry = jax.lax.fori_loop(2, 7, outer_a, carry)
        cp_in.wait()
        carry = jax.lax.fori_loop(7, _TH // 16, outer_a, carry)
        pltpu.sync_copy(ind_a, ind_hbm.at[wid, pl.ds(0, _TH)])
        carry = jax.lax.fori_loop(_TH // 16, _T // 16, outer_b, carry)
        pltpu.sync_copy(ind_b, ind_hbm.at[wid, pl.ds(_TH, _T - _TH)])
        s_v[...] = carry[3]
        pltpu.sync_copy(s_v, stats_hbm.at[wid])


_scan = functools.partial(
    pl.kernel,
    out_type=[
        jax.ShapeDtypeStruct((_B, _T), jnp.int32),
        jax.ShapeDtypeStruct((_B, 16), jnp.float32),
    ],
    mesh=plsc.VectorSubcoreMesh(core_axis_name="c", subcore_axis_name="s",
                                num_cores=1),
    compiler_params=pltpu.CompilerParams(
        use_tc_tiling_on_sc=False, needs_layout_passes=False),
    scratch_types=[
        pltpu.VMEM((_T + 2, _NC), jnp.float32),
        pltpu.VMEM((_TH,), jnp.int32),
        pltpu.VMEM((_TH,), jnp.int32),
        pltpu.VMEM((16,), jnp.float32),
        pltpu.SemaphoreType.DMA,
        pltpu.SemaphoreType.DMA,
        pltpu.SemaphoreType.DMA,
    ],
)(_scan_body)


def _zq_body(ind_ref, w_ref, st_ref, zq_ref, loss_ref, v_ref):
    pid = pl.program_id(0)
    ind = ind_ref[...]
    w = w_ref[...]
    iot = jax.lax.broadcasted_iota(jnp.int32, (_BM, _NC), 1)
    oh = jnp.where(iot == ind, 1.0, 0.0).astype(jnp.float32)
    zq_ref[...] = jax.lax.dot_general(oh, w, (((1,), (0,)), ((), ())),
                                      precision=jax.lax.Precision.HIGHEST,
                                      preferred_element_type=jnp.float32)
    pmax = jnp.max(ind)

    @pl.when(pid == 0)
    def _():
        v_ref[0, 0] = pmax

    @pl.when(pid != 0)
    def _():
        v_ref[0, 0] = jnp.maximum(v_ref[0, 0], pmax)

    @pl.when(pid == _GRID - 1)
    def _():
        m = jnp.sum(st_ref[:, 0:1]) * jnp.float32(1.0 / (_B * _T * _D))
        loss_ref[0, 0] = jnp.float32(_BETA) * m + m


_zq = pl.pallas_call(
    _zq_body,
    grid=(_GRID,),
    in_specs=[
        pl.BlockSpec((_BM, 1), lambda i: (i, 0)),
        pl.BlockSpec((_NC, _D), lambda i: (0, 0)),
        pl.BlockSpec((_B, 16), lambda i: (0, 0)),
    ],
    out_specs=[
        pl.BlockSpec((_BM, _D), lambda i: (i, 0)),
        pl.BlockSpec((1, 1), lambda i: (0, 0), memory_space=pltpu.SMEM),
        pl.BlockSpec((1, 1), lambda i: (0, 0), memory_space=pltpu.SMEM),
    ],
    out_shape=[
        jax.ShapeDtypeStruct((_B * _T, _D), jnp.float32),
        jax.ShapeDtypeStruct((1, 1), jnp.float32),
        jax.ShapeDtypeStruct((1, 1), jnp.int32),
    ],
)


def kernel(z, weight):
    zf = z.reshape(_B * _T, _D)
    d = _dist(zf, weight)
    ind, stats = _scan(d.reshape(_B, _T, _NC))
    zq, loss, vmax = _zq(ind.reshape(_B * _T, 1), weight, stats)
    return (zq.reshape(z.shape), loss.reshape(()), ind, vmax.reshape(()))
